# baseline (speedup 1.0000x reference)
import functools

import jax
import jax.numpy as jnp
from jax import lax
from jax.experimental import pallas as pl
from jax.experimental.pallas import tpu as pltpu
from jax.experimental.pallas import tpu_sc as plsc

_BATCH = 4096
_EMBED = 64


@functools.lru_cache(maxsize=None)
def _build(n_users, n_items):
    info = plsc.get_sparse_core_info()
    nw = info.num_cores * info.num_subcores
    nc = info.num_cores
    b_per_w = _BATCH // nw

    mesh = plsc.VectorSubcoreMesh(core_axis_name="c", subcore_axis_name="s")

    @functools.partial(
        pl.kernel,
        mesh=mesh,
        compiler_params=pltpu.CompilerParams(use_tc_tiling_on_sc=False),
        out_type=(
            jax.ShapeDtypeStruct((_BATCH, _EMBED), jnp.float32),
            jax.ShapeDtypeStruct((_BATCH, _EMBED), jnp.float32),
        ),
        scratch_types=[
            pltpu.VMEM((b_per_w,), jnp.int32),
            pltpu.VMEM((b_per_w,), jnp.int32),
            pltpu.VMEM((b_per_w, _EMBED), jnp.float32),
            pltpu.VMEM((b_per_w, _EMBED), jnp.float32),
            pltpu.SemaphoreType.DMA,
            pltpu.SemaphoreType.DMA,
        ],
    )
    def gather_kernel(user_hbm, item_hbm, utab_hbm, itab_hbm,
                      uout_hbm, iout_hbm,
                      uidx_v, iidx_v, urows_v, irows_v, usem, isem):
        wid = lax.axis_index("s") * nc + lax.axis_index("c")
        base = wid * b_per_w
        pltpu.sync_copy(user_hbm.at[pl.ds(base, b_per_w)], uidx_v)
        pltpu.sync_copy(item_hbm.at[pl.ds(base, b_per_w)], iidx_v)
        ucopy = pltpu.async_copy(utab_hbm.at[uidx_v], urows_v, usem)
        icopy = pltpu.async_copy(itab_hbm.at[iidx_v], irows_v, isem)
        ucopy.wait()
        pltpu.sync_copy(urows_v, uout_hbm.at[pl.ds(base, b_per_w)])
        icopy.wait()
        pltpu.sync_copy(irows_v, iout_hbm.at[pl.ds(base, b_per_w)])

    return gather_kernel


def kernel(user, item, user_table, item_table):
    gather_kernel = _build(user_table.shape[0], item_table.shape[0])
    user_emb, item_emb = gather_kernel(
        user.astype(jnp.int32), item.astype(jnp.int32),
        user_table, item_table)
    return (user_emb, item_emb)
